# baseline (speedup 1.0000x reference)
import functools

import jax
import jax.numpy as jnp
from jax import lax
from jax.experimental import pallas as pl
from jax.experimental.pallas import tpu as pltpu
from jax.experimental.pallas import tpu_sc as plsc

_N = 10000
_F = 128
_H = 16
_C = 10
_NP = 10240
_EP = 327680
_B = 128
_NW = 32
_EPW = _EP // _NW
_NBLK = _EPW // _B
_RPT = _NP // 16

_f32 = jnp.float32
_i32 = jnp.int32


def _sc_prop(hn, hnd, scale, src, dst4):
    mesh = plsc.VectorSubcoreMesh(core_axis_name="c", subcore_axis_name="s")

    @functools.partial(
        pl.kernel,
        mesh=mesh,
        compiler_params=pltpu.CompilerParams(use_tc_tiling_on_sc=False),
        out_type=[
            jax.ShapeDtypeStruct((2, _NP, _H), _f32),
            jax.ShapeDtypeStruct((2, _NP), _f32),
        ],
        scratch_types=(
            [
                pltpu.VMEM((_EPW,), _i32),
                pltpu.VMEM((_NBLK, 1, _B), _i32),
            ]
            + [pltpu.VMEM((_B, _H), _f32)] * 4
            + [pltpu.VMEM((_B,), _f32)] * 4
            + [pltpu.VMEM((_B, _H), _f32)] * 2
            + [
                pltpu.VMEM((_RPT, _H), _f32),
                pltpu.VMEM((_RPT,), _f32),
                pltpu.VMEM_SHARED((_NP, _H), _f32),
                pltpu.VMEM_SHARED((_NP,), _f32),
            ]
            + [pltpu.SemaphoreType.DMA] * 10
        ),
    )
    def k(hn_h, hnd_h, scale_h, src_h, dst4_h, acc_o, s_o,
          ixs_all, ixd_all, rs0, rd0, rs1, rd1,
          sc0, sc1, eb0, eb1, wr0, wr1,
          zb2, zb1, acc_sh, s_sh, *sems):
        cid = lax.axis_index("c")
        sid = lax.axis_index("s")
        wid = sid * 2 + cid
        zv = jnp.zeros((_H,), _f32)
        pltpu.sync_copy(src_h.at[pl.ds(wid * _EPW, _EPW)], ixs_all)
        pltpu.sync_copy(dst4_h.at[wid], ixd_all)

        def zrow(i, carry):
            zb2[i, :] = zv
            return carry

        lax.fori_loop(0, _RPT, zrow, 0)

        def zrow1(i, carry):
            zb1[pl.ds(i * 16, 16)] = zv
            return carry

        lax.fori_loop(0, _RPT // 16, zrow1, 0)
        pltpu.sync_copy(zb2, acc_sh.at[pl.ds(sid * _RPT, _RPT), :])
        pltpu.sync_copy(zb1, s_sh.at[pl.ds(sid * _RPT, _RPT)])
        plsc.subcore_barrier()

        ii0 = lax.iota(_i32, 16)
        dn = jax.lax.GatherDimensionNumbers(
            offset_dims=(), collapsed_slice_dims=(0,), start_index_map=(0,))
        rot_idx = [((ii0 + sh) % 16)[:, None] for sh in (8, 4, 2, 1)]

        def lane_sum(x):
            for ridx in rot_idx:
                x = x + jax.lax.gather(
                    x, ridx, dn, (1,),
                    mode=jax.lax.GatherScatterMode.PROMISE_IN_BOUNDS)
            return x

        def compute(rows_s, rows_d, scale_s, e_buf, w_rows):
            for g in range(_B // 16):
                sl = pl.ds(g * 16, 16)
                s16 = scale_s[sl]
                e16 = jnp.zeros((16,), _f32)
                for j in range(16):
                    jj = g * 16 + j
                    a = rows_s[jj, :]
                    ev = jnp.exp(lane_sum(a * rows_d[jj, :]))
                    e16 = jnp.where(ii0 == j, ev, e16)
                    ew = ev * s16
                    w_rows[jj, :] = a * jnp.broadcast_to(ew[j:j + 1], (_H,))
                e_buf[sl] = e16

        rs = (rs0, rs1)
        rd = (rd0, rd1)
        sc = (sc0, sc1)
        eb = (eb0, eb1)
        wr = (wr0, wr1)

        def gathers(blk, p, sem_base):
            six = ixs_all.at[pl.ds(blk * _B, _B)]
            dix = ixd_all.at[blk, 0]
            return (
                pltpu.async_copy(hn_h.at[six], rs[p], sems[sem_base]),
                pltpu.async_copy(hnd_h.at[dix], rd[p], sems[sem_base + 1]),
                pltpu.async_copy(scale_h.at[six], sc[p], sems[sem_base + 2]),
            )

        def superblock(sb, carry):
            blk_a = sb * 2
            blk_b = blk_a + 1
            ga = gathers(blk_a, 0, 0)
            gb = gathers(blk_b, 1, 3)

            @pl.when(sb > 0)
            def _drain_prev():
                pltpu.make_async_copy(
                    wr1, acc_sh.at[ixd_all.at[blk_a - 1, 0]], sems[8]).wait()
                pltpu.make_async_copy(
                    eb1, s_sh.at[ixd_all.at[blk_a - 1, 0]], sems[9]).wait()

            for c in ga:
                c.wait()
            compute(rs0, rd0, sc0, eb0, wr0)
            sa1 = pltpu.async_copy(
                wr0, acc_sh.at[ixd_all.at[blk_a, 0]], sems[6], add=True)
            sa2 = pltpu.async_copy(
                eb0, s_sh.at[ixd_all.at[blk_a, 0]], sems[7], add=True)
            for c in gb:
                c.wait()
            compute(rs1, rd1, sc1, eb1, wr1)
            sa1.wait()
            sa2.wait()
            pltpu.async_copy(
                wr1, acc_sh.at[ixd_all.at[blk_b, 0]], sems[8], add=True)
            pltpu.async_copy(
                eb1, s_sh.at[ixd_all.at[blk_b, 0]], sems[9], add=True)
            return carry

        lax.fori_loop(0, _NBLK // 2, superblock, 0)
        pltpu.make_async_copy(
            wr1, acc_sh.at[ixd_all.at[_NBLK - 1, 0]], sems[8]).wait()
        pltpu.make_async_copy(
            eb1, s_sh.at[ixd_all.at[_NBLK - 1, 0]], sems[9]).wait()
        plsc.subcore_barrier()
        pltpu.sync_copy(acc_sh.at[pl.ds(sid * _RPT, _RPT), :],
                        acc_o.at[cid, pl.ds(sid * _RPT, _RPT), :])
        pltpu.sync_copy(s_sh.at[pl.ds(sid * _RPT, _RPT)],
                        s_o.at[cid, pl.ds(sid * _RPT, _RPT)])

    return k(hn, hnd, scale, src, dst4)


_RB = 1000
_NG = _N // _RB


def _tc_head(x, w1t, b1):
    def body(x_ref, w_ref, b_ref, hn_ref, sc_ref):
        h = jnp.maximum(
            jnp.dot(x_ref[...], w_ref[...],
                    preferred_element_type=_f32,
                    precision=lax.Precision.HIGHEST) + b_ref[...], 0.0)
        rn = jnp.sqrt(jnp.sum(h * h, axis=1, keepdims=True)) + 1e-12
        hn_ref[...] = h / rn
        sc_ref[...] = rn

    return pl.pallas_call(
        body,
        grid=(_NG,),
        in_specs=[pl.BlockSpec((_RB, _F), lambda i: (i, 0)),
                  pl.BlockSpec((_F, _H), lambda i: (0, 0)),
                  pl.BlockSpec((1, _H), lambda i: (0, 0))],
        out_specs=[pl.BlockSpec((_RB, _H), lambda i: (i, 0)),
                   pl.BlockSpec((_RB, 1), lambda i: (i, 0))],
        out_shape=[jax.ShapeDtypeStruct((_N, _H), _f32),
                   jax.ShapeDtypeStruct((_N, 1), _f32)],
    )(x, w1t, b1)


def _combine_block(acc_ref, s_ref, hn_ref, sc_ref, beta):
    hnv = hn_ref[...]
    es = jnp.exp(beta * jnp.sum(hnv * hnv, axis=1, keepdims=True))
    h = hnv * sc_ref[...]
    num = acc_ref[0] + acc_ref[1] + es * h
    den = s_ref[0] + s_ref[1] + es
    return num / den


def _tc_combine(acc, s3, hn, scl, beta2v):
    def body(acc_ref, s_ref, hn_ref, sc_ref, b2_ref, hn2_ref, hnd2_ref, sc2_ref):
        out = _combine_block(acc_ref, s_ref, hn_ref, sc_ref, 1.0)
        rn = jnp.sqrt(jnp.sum(out * out, axis=1, keepdims=True)) + 1e-12
        hn2 = out / rn
        hn2_ref[...] = hn2
        hnd2_ref[...] = hn2 * b2_ref[0, 0]
        sc2_ref[...] = rn

    return pl.pallas_call(
        body,
        grid=(_NG,),
        in_specs=[pl.BlockSpec((2, _RB, _H), lambda i: (0, i, 0)),
                  pl.BlockSpec((2, _RB, 1), lambda i: (0, i, 0)),
                  pl.BlockSpec((_RB, _H), lambda i: (i, 0)),
                  pl.BlockSpec((_RB, 1), lambda i: (i, 0)),
                  pl.BlockSpec((1, 1), lambda i: (0, 0))],
        out_specs=[pl.BlockSpec((_RB, _H), lambda i: (i, 0)),
                   pl.BlockSpec((_RB, _H), lambda i: (i, 0)),
                   pl.BlockSpec((_RB, 1), lambda i: (i, 0))],
        out_shape=[jax.ShapeDtypeStruct((_N, _H), _f32),
                   jax.ShapeDtypeStruct((_N, _H), _f32),
                   jax.ShapeDtypeStruct((_N, 1), _f32)],
    )(acc, s3, hn, scl, beta2v)


def _tc_tail(acc, s3, hn, scl, beta2v, w2t, b2):
    def body(acc_ref, s_ref, hn_ref, sc_ref, b2v_ref, w2_ref, b2_ref, out_ref):
        out = _combine_block(acc_ref, s_ref, hn_ref, sc_ref, b2v_ref[0, 0])
        logits = jnp.dot(out, w2_ref[...],
                         preferred_element_type=_f32,
                         precision=lax.Precision.HIGHEST) + b2_ref[...]
        m = jnp.max(logits, axis=1, keepdims=True)
        lse = jnp.log(jnp.sum(jnp.exp(logits - m), axis=1, keepdims=True)) + m
        out_ref[...] = logits - lse

    return pl.pallas_call(
        body,
        grid=(_NG,),
        in_specs=[pl.BlockSpec((2, _RB, _H), lambda i: (0, i, 0)),
                  pl.BlockSpec((2, _RB, 1), lambda i: (0, i, 0)),
                  pl.BlockSpec((_RB, _H), lambda i: (i, 0)),
                  pl.BlockSpec((_RB, 1), lambda i: (i, 0)),
                  pl.BlockSpec((1, 1), lambda i: (0, 0)),
                  pl.BlockSpec((_H, _C), lambda i: (0, 0)),
                  pl.BlockSpec((1, _C), lambda i: (0, 0))],
        out_specs=pl.BlockSpec((_RB, _C), lambda i: (i, 0)),
        out_shape=jax.ShapeDtypeStruct((_N, _C), _f32),
    )(acc, s3, hn, scl, beta2v, w2t, b2)


def kernel(x, edge_index, W1, b1, beta2, W2, b2):
    x = x.astype(_f32)
    src = edge_index[0].astype(_i32)
    dst = edge_index[1].astype(_i32)
    fill = jnp.full((_EP - src.shape[0],), _N, _i32)
    srcp = jnp.concatenate([src, fill])
    dst4 = jnp.concatenate([dst, fill]).reshape(_NW, _NBLK, 1, _B)
    beta2v = beta2.reshape(1, 1).astype(_f32)

    hn1, scl1 = _tc_head(x, W1.T.astype(_f32), b1.reshape(1, _H).astype(_f32))
    hn1p = jnp.pad(hn1, ((0, _NP - _N), (0, 0)))
    scl1p = jnp.pad(scl1[:, 0], (0, _NP - _N))
    acc1, s1 = _sc_prop(hn1p, hn1p, scl1p, srcp, dst4)

    hn2, hnd2, scl2 = _tc_combine(acc1[:, :_N, :], s1[:, :_N, None],
                                  hn1, scl1, beta2v)
    hn2p = jnp.pad(hn2, ((0, _NP - _N), (0, 0)))
    hnd2p = jnp.pad(hnd2, ((0, _NP - _N), (0, 0)))
    scl2p = jnp.pad(scl2[:, 0], (0, _NP - _N))
    acc2, s2 = _sc_prop(hn2p, hnd2p, scl2p, srcp, dst4)

    return _tc_tail(acc2[:, :_N, :], s2[:, :_N, None], hn2, scl2, beta2v,
                    W2.T.astype(_f32), b2.reshape(1, _C).astype(_f32))


# speedup vs baseline: 1.3050x; 1.3050x over previous
import functools

import jax
import jax.numpy as jnp
from jax import lax
from jax.experimental import pallas as pl
from jax.experimental.pallas import tpu as pltpu
from jax.experimental.pallas import tpu_sc as plsc

_N = 10000
_F = 128
_H = 16
_C = 10
_NP = 10240
_EP = 327680
_B = 128
_NW = 32
_EPW = _EP // _NW
_NBLK = _EPW // _B
_RPT = _NP // 16

_f32 = jnp.float32
_i32 = jnp.int32


def _sc_prop(hn, hnd, scale, src, dst4):
    mesh = plsc.VectorSubcoreMesh(core_axis_name="c", subcore_axis_name="s")

    @functools.partial(
        pl.kernel,
        mesh=mesh,
        compiler_params=pltpu.CompilerParams(use_tc_tiling_on_sc=False),
        out_type=[
            jax.ShapeDtypeStruct((2, _NP, _H), _f32),
            jax.ShapeDtypeStruct((2, _NP), _f32),
        ],
        scratch_types=(
            [
                pltpu.VMEM((_EPW,), _i32),
                pltpu.VMEM((_NBLK, 1, _B), _i32),
            ]
            + [pltpu.VMEM((_B, _H), _f32)] * 8
            + [pltpu.VMEM((_B,), _f32)] * 8
            + [pltpu.VMEM((_B, _H), _f32)] * 4
            + [
                pltpu.VMEM((_RPT, _H), _f32),
                pltpu.VMEM((_RPT,), _f32),
                pltpu.VMEM_SHARED((_NP, _H), _f32),
                pltpu.VMEM_SHARED((_NP,), _f32),
            ]
            + [pltpu.SemaphoreType.DMA] * 20
        ),
    )
    def k(hn_h, hnd_h, scale_h, src_h, dst4_h, acc_o, s_o,
          ixs_all, ixd_all, rs0, rd0, rs1, rd1, rs2, rd2, rs3, rd3,
          sc0, sc1, sc2, sc3, eb0, eb1, eb2, eb3, wr0, wr1, wr2, wr3,
          zb2, zb1, acc_sh, s_sh, *sems):
        cid = lax.axis_index("c")
        sid = lax.axis_index("s")
        wid = sid * 2 + cid
        zv = jnp.zeros((_H,), _f32)
        pltpu.sync_copy(src_h.at[pl.ds(wid * _EPW, _EPW)], ixs_all)
        pltpu.sync_copy(dst4_h.at[wid], ixd_all)

        def zrow(i, carry):
            zb2[i, :] = zv
            return carry

        lax.fori_loop(0, _RPT, zrow, 0)

        def zrow1(i, carry):
            zb1[pl.ds(i * 16, 16)] = zv
            return carry

        lax.fori_loop(0, _RPT // 16, zrow1, 0)
        pltpu.sync_copy(zb2, acc_sh.at[pl.ds(sid * _RPT, _RPT), :])
        pltpu.sync_copy(zb1, s_sh.at[pl.ds(sid * _RPT, _RPT)])
        plsc.subcore_barrier()

        ii0 = lax.iota(_i32, 16)
        dn = jax.lax.GatherDimensionNumbers(
            offset_dims=(), collapsed_slice_dims=(0,), start_index_map=(0,))
        rot_idx = [((ii0 + sh) % 16)[:, None] for sh in (8, 4, 2, 1)]

        def lane_sum(x):
            for ridx in rot_idx:
                x = x + jax.lax.gather(
                    x, ridx, dn, (1,),
                    mode=jax.lax.GatherScatterMode.PROMISE_IN_BOUNDS)
            return x

        def compute(rows_s, rows_d, scale_s, e_buf, w_rows):
            def grp(g, carry):
                sl = pl.ds(g * 16, 16)
                s16 = scale_s[sl]
                e16 = jnp.zeros((16,), _f32)
                for j in range(16):
                    jj = g * 16 + j
                    a = rows_s[jj, :]
                    ev = jnp.exp(lane_sum(a * rows_d[jj, :]))
                    e16 = jnp.where(ii0 == j, ev, e16)
                    ew = ev * s16
                    w_rows[jj, :] = a * jnp.broadcast_to(ew[j:j + 1], (_H,))
                e_buf[sl] = e16
                return carry

            lax.fori_loop(0, _B // 16, grp, 0)

        rs = (rs0, rs1, rs2, rs3)
        rd = (rd0, rd1, rd2, rd3)
        sc = (sc0, sc1, sc2, sc3)
        eb = (eb0, eb1, eb2, eb3)
        wr = (wr0, wr1, wr2, wr3)

        def gathers(blk, p):
            six = ixs_all.at[pl.ds(blk * _B, _B)]
            dix = ixd_all.at[blk, 0]
            return (
                pltpu.async_copy(hn_h.at[six], rs[p], sems[3 * p]),
                pltpu.async_copy(hnd_h.at[dix], rd[p], sems[3 * p + 1]),
                pltpu.async_copy(scale_h.at[six], sc[p], sems[3 * p + 2]),
            )

        def superblock(sb, carry):
            blk0 = sb * 4
            g = [gathers(blk0 + p, p) for p in range(4)]
            scats = []
            for p in range(4):
                for c in g[p]:
                    c.wait()
                compute(rs[p], rd[p], sc[p], eb[p], wr[p])
                dix = ixd_all.at[blk0 + p, 0]
                scats.append(pltpu.async_copy(
                    wr[p], acc_sh.at[dix], sems[12 + 2 * p], add=True))
                scats.append(pltpu.async_copy(
                    eb[p], s_sh.at[dix], sems[13 + 2 * p], add=True))
            for c in scats:
                c.wait()
            return carry

        lax.fori_loop(0, _NBLK // 4, superblock, 0)
        plsc.subcore_barrier()
        pltpu.sync_copy(acc_sh.at[pl.ds(sid * _RPT, _RPT), :],
                        acc_o.at[cid, pl.ds(sid * _RPT, _RPT), :])
        pltpu.sync_copy(s_sh.at[pl.ds(sid * _RPT, _RPT)],
                        s_o.at[cid, pl.ds(sid * _RPT, _RPT)])

    return k(hn, hnd, scale, src, dst4)


_RB = 1000
_NG = _N // _RB


def _tc_head(x, w1t, b1):
    def body(x_ref, w_ref, b_ref, hn_ref, sc_ref):
        h = jnp.maximum(
            jnp.dot(x_ref[...], w_ref[...],
                    preferred_element_type=_f32,
                    precision=lax.Precision.HIGHEST) + b_ref[...], 0.0)
        rn = jnp.sqrt(jnp.sum(h * h, axis=1, keepdims=True)) + 1e-12
        hn_ref[...] = h / rn
        sc_ref[...] = rn

    return pl.pallas_call(
        body,
        grid=(_NG,),
        in_specs=[pl.BlockSpec((_RB, _F), lambda i: (i, 0)),
                  pl.BlockSpec((_F, _H), lambda i: (0, 0)),
                  pl.BlockSpec((1, _H), lambda i: (0, 0))],
        out_specs=[pl.BlockSpec((_RB, _H), lambda i: (i, 0)),
                   pl.BlockSpec((_RB, 1), lambda i: (i, 0))],
        out_shape=[jax.ShapeDtypeStruct((_N, _H), _f32),
                   jax.ShapeDtypeStruct((_N, 1), _f32)],
    )(x, w1t, b1)


def _combine_block(acc_ref, s_ref, hn_ref, sc_ref, beta):
    hnv = hn_ref[...]
    es = jnp.exp(beta * jnp.sum(hnv * hnv, axis=1, keepdims=True))
    h = hnv * sc_ref[...]
    num = acc_ref[0] + acc_ref[1] + es * h
    den = s_ref[0] + s_ref[1] + es
    return num / den


def _tc_combine(acc, s3, hn, scl, beta2v):
    def body(acc_ref, s_ref, hn_ref, sc_ref, b2_ref, hn2_ref, hnd2_ref, sc2_ref):
        out = _combine_block(acc_ref, s_ref, hn_ref, sc_ref, 1.0)
        rn = jnp.sqrt(jnp.sum(out * out, axis=1, keepdims=True)) + 1e-12
        hn2 = out / rn
        hn2_ref[...] = hn2
        hnd2_ref[...] = hn2 * b2_ref[0, 0]
        sc2_ref[...] = rn

    return pl.pallas_call(
        body,
        grid=(_NG,),
        in_specs=[pl.BlockSpec((2, _RB, _H), lambda i: (0, i, 0)),
                  pl.BlockSpec((2, _RB, 1), lambda i: (0, i, 0)),
                  pl.BlockSpec((_RB, _H), lambda i: (i, 0)),
                  pl.BlockSpec((_RB, 1), lambda i: (i, 0)),
                  pl.BlockSpec((1, 1), lambda i: (0, 0))],
        out_specs=[pl.BlockSpec((_RB, _H), lambda i: (i, 0)),
                   pl.BlockSpec((_RB, _H), lambda i: (i, 0)),
                   pl.BlockSpec((_RB, 1), lambda i: (i, 0))],
        out_shape=[jax.ShapeDtypeStruct((_N, _H), _f32),
                   jax.ShapeDtypeStruct((_N, _H), _f32),
                   jax.ShapeDtypeStruct((_N, 1), _f32)],
    )(acc, s3, hn, scl, beta2v)


def _tc_tail(acc, s3, hn, scl, beta2v, w2t, b2):
    def body(acc_ref, s_ref, hn_ref, sc_ref, b2v_ref, w2_ref, b2_ref, out_ref):
        out = _combine_block(acc_ref, s_ref, hn_ref, sc_ref, b2v_ref[0, 0])
        logits = jnp.dot(out, w2_ref[...],
                         preferred_element_type=_f32,
                         precision=lax.Precision.HIGHEST) + b2_ref[...]
        m = jnp.max(logits, axis=1, keepdims=True)
        lse = jnp.log(jnp.sum(jnp.exp(logits - m), axis=1, keepdims=True)) + m
        out_ref[...] = logits - lse

    return pl.pallas_call(
        body,
        grid=(_NG,),
        in_specs=[pl.BlockSpec((2, _RB, _H), lambda i: (0, i, 0)),
                  pl.BlockSpec((2, _RB, 1), lambda i: (0, i, 0)),
                  pl.BlockSpec((_RB, _H), lambda i: (i, 0)),
                  pl.BlockSpec((_RB, 1), lambda i: (i, 0)),
                  pl.BlockSpec((1, 1), lambda i: (0, 0)),
                  pl.BlockSpec((_H, _C), lambda i: (0, 0)),
                  pl.BlockSpec((1, _C), lambda i: (0, 0))],
        out_specs=pl.BlockSpec((_RB, _C), lambda i: (i, 0)),
        out_shape=jax.ShapeDtypeStruct((_N, _C), _f32),
    )(acc, s3, hn, scl, beta2v, w2t, b2)


def kernel(x, edge_index, W1, b1, beta2, W2, b2):
    x = x.astype(_f32)
    src = edge_index[0].astype(_i32)
    dst = edge_index[1].astype(_i32)
    fill = jnp.full((_EP - src.shape[0],), _N, _i32)
    srcp = jnp.concatenate([src, fill])
    dst4 = jnp.concatenate([dst, fill]).reshape(_NW, _NBLK, 1, _B)
    beta2v = beta2.reshape(1, 1).astype(_f32)

    hn1, scl1 = _tc_head(x, W1.T.astype(_f32), b1.reshape(1, _H).astype(_f32))
    hn1p = jnp.pad(hn1, ((0, _NP - _N), (0, 0)))
    scl1p = jnp.pad(scl1[:, 0], (0, _NP - _N))
    acc1, s1 = _sc_prop(hn1p, hn1p, scl1p, srcp, dst4)

    hn2, hnd2, scl2 = _tc_combine(acc1[:, :_N, :], s1[:, :_N, None],
                                  hn1, scl1, beta2v)
    hn2p = jnp.pad(hn2, ((0, _NP - _N), (0, 0)))
    hnd2p = jnp.pad(hnd2, ((0, _NP - _N), (0, 0)))
    scl2p = jnp.pad(scl2[:, 0], (0, _NP - _N))
    acc2, s2 = _sc_prop(hn2p, hnd2p, scl2p, srcp, dst4)

    return _tc_tail(acc2[:, :_N, :], s2[:, :_N, None], hn2, scl2, beta2v,
                    W2.T.astype(_f32), b2.reshape(1, _C).astype(_f32))
